# baseline (speedup 1.0000x reference)
import functools

import jax
import jax.numpy as jnp
from jax import lax
from jax.experimental import pallas as pl
from jax.experimental.pallas import tpu as pltpu
from jax.experimental.pallas import tpu_sc as plsc

NC = 2
NS = 16
NW = NC * NS
CH = 40
R = 6
IB = 2 * R
RCH = 400


def _sc_mesh():
    return plsc.VectorSubcoreMesh(core_axis_name="c", subcore_axis_name="s")


def _rowchunk_copy(s, n, copy_one):
    nchunk = n // RCH
    reps = (nchunk + NS - 1) // NS
    for rep in range(reps):
        cid = s + rep * NS
        if (rep + 1) * NS <= nchunk:
            copy_one(cid)
        else:
            @pl.when(cid < nchunk)
            def _():
                copy_one(cid)


def _deg_call(ei_flat, zeros16, n, e):
    CHD = 80
    P = 5
    epw = e // NW
    nch = epw // CHD
    ones_rows = jnp.concatenate(
        [jnp.ones((CHD, 1), jnp.float32), jnp.zeros((CHD, 15), jnp.float32)],
        axis=1)

    @functools.partial(
        pl.kernel,
        out_type=jax.ShapeDtypeStruct((NC * n, 16), jnp.float32),
        mesh=_sc_mesh(),
        scratch_types=(
            [pltpu.VMEM((2 * P, CHD), jnp.int32),
             pltpu.VMEM((CHD, 16), jnp.float32),
             pltpu.VMEM_SHARED((n, 16), jnp.float32)]
            + [pltpu.SemaphoreType.DMA] * (3 * P)
        ),
    )
    def deg_kernel(dst_hbm, ones_hbm, z16_hbm, out_hbm, di_v, ones_v, dacc,
                   *sems):
        sidx = sems[:2 * P]
        ssc = sems[2 * P:]
        c = lax.axis_index("c")
        s = lax.axis_index("s")
        wid = c * NS + s
        base = e + wid * epw
        _rowchunk_copy(s, n, lambda cid: pltpu.sync_copy(
            z16_hbm.at[pl.ds(cid * RCH, RCH)],
            dacc.at[pl.ds(cid * RCH, RCH)]))
        pltpu.sync_copy(ones_hbm, ones_v)
        plsc.subcore_barrier()

        def idx_start(j, slot):
            pltpu.async_copy(dst_hbm.at[pl.ds(base + j * CHD, CHD)],
                             di_v.at[slot], sidx[slot])

        def idx_wait(j, slot):
            pltpu.make_async_copy(dst_hbm.at[pl.ds(base + j * CHD, CHD)],
                                  di_v.at[slot], sidx[slot]).wait()

        def scat_start(slot, b):
            pltpu.async_copy(ones_v, dacc.at[di_v.at[slot]], ssc[b],
                             add=True)

        def scat_wait(slot, b):
            pltpu.make_async_copy(ones_v, dacc.at[di_v.at[slot]],
                                  ssc[b]).wait()

        def step(j, u, do_wait, do_issue):
            b = u % P
            if do_wait:
                scat_wait((u + P) % (2 * P), b)
            if do_issue:
                idx_start(j + P, (u + P) % (2 * P))
            idx_wait(j, u)
            scat_start(u, b)

        for i in range(P):
            idx_start(i, i)
        for j in range(P):
            step(j, j, False, True)

        def body(k, carry):
            for u2 in range(2 * P):
                j = P + k * 2 * P + u2
                step(j, (P + u2) % (2 * P), True, True)
            return carry

        n_main = (nch - 2 * P - P) // (2 * P)
        lax.fori_loop(0, n_main, body, 0)
        for i in range(2 * P):
            j = nch - 2 * P + i
            step(j, j % (2 * P), True, i < P)
        for i in range(P):
            j = nch - P + i
            scat_wait(j % (2 * P), j % P)
        plsc.subcore_barrier()
        _rowchunk_copy(s, n, lambda cid: pltpu.sync_copy(
            dacc.at[pl.ds(cid * RCH, RCH)],
            out_hbm.at[pl.ds(c * n + cid * RCH, RCH)]))

    return deg_kernel(ei_flat, ones_rows, zeros16)


def _edge_call(hs, ei_flat, zeros, n, d, e):
    epw = e // NW
    nch = epw // CH

    @functools.partial(
        pl.kernel,
        out_type=jax.ShapeDtypeStruct((NC * n, d), jnp.float32),
        mesh=_sc_mesh(),
        scratch_types=(
            [pltpu.VMEM((IB, CH), jnp.int32),
             pltpu.VMEM((IB, CH), jnp.int32),
             pltpu.VMEM((R, CH, d), jnp.float32),
             pltpu.VMEM_SHARED((n, d), jnp.float32)]
            + [pltpu.SemaphoreType.DMA] * (IB + R)
        ),
    )
    def edge_kernel(hs_hbm, ei_hbm, z_hbm, out_hbm,
                    si_v, di_v, rows_v, acc, *sems):
        sidx = sems[:IB]
        srow = sems[IB:]
        c = lax.axis_index("c")
        s = lax.axis_index("s")
        wid = c * NS + s
        sbase = wid * epw
        dbase = e + wid * epw
        _rowchunk_copy(s, n, lambda cid: pltpu.sync_copy(
            z_hbm.at[pl.ds(cid * RCH, RCH)],
            acc.at[pl.ds(cid * RCH, RCH)]))
        plsc.subcore_barrier()

        def idx_start(j, u):
            pltpu.async_copy(ei_hbm.at[pl.ds(sbase + j * CH, CH)],
                             si_v.at[u], sidx[u])
            pltpu.async_copy(ei_hbm.at[pl.ds(dbase + j * CH, CH)],
                             di_v.at[u], sidx[u])

        def idx_wait(j, u):
            pltpu.make_async_copy(ei_hbm.at[pl.ds(sbase + j * CH, CH)],
                                  si_v.at[u], sidx[u]).wait()
            pltpu.make_async_copy(ei_hbm.at[pl.ds(dbase + j * CH, CH)],
                                  di_v.at[u], sidx[u]).wait()

        def gather_start(islot, b):
            pltpu.async_copy(hs_hbm.at[si_v.at[islot]], rows_v.at[b], srow[b])

        def gather_wait(islot, b):
            pltpu.make_async_copy(hs_hbm.at[si_v.at[islot]], rows_v.at[b],
                                  srow[b]).wait()

        def scat(islot, b):
            pltpu.sync_copy(rows_v.at[b], acc.at[di_v.at[islot]], add=True)

        for i in range(IB):
            idx_start(i, i)
        for b in range(R):
            idx_wait(b, b)
            gather_start(b, b)

        def step(j, u, do_idx, do_gather):
            b = u % R
            islot = u % IB
            islot2 = (u + R) % IB
            gather_wait(islot, b)
            scat(islot, b)
            if do_idx:
                idx_start(j + IB, islot)
            if do_gather:
                idx_wait(j + R, islot2)
                gather_start(islot2, b)

        def body(k, carry):
            for u in range(IB):
                step(k * IB + u, u, True, True)
            return carry

        n_main = (nch - IB) // IB
        lax.fori_loop(0, n_main, body, 0)
        for j in range(n_main * IB, nch):
            step(j, j % IB, j + IB < nch, j + R < nch)

        plsc.subcore_barrier()
        _rowchunk_copy(s, n, lambda cid: pltpu.sync_copy(
            acc.at[pl.ds(cid * RCH, RCH)],
            out_hbm.at[pl.ds(c * n + cid * RCH, RCH)]))

    return edge_kernel(hs, ei_flat, zeros)


def _hcat_call(x, wcat, bcat, n, d):
    blk = 2000

    def body(x_ref, w_ref, b_ref, h_ref):
        h_ref[...] = jnp.dot(x_ref[...], w_ref[...],
                             preferred_element_type=jnp.float32) + b_ref[...]

    return pl.pallas_call(
        body,
        grid=(n // blk,),
        in_specs=[
            pl.BlockSpec((blk, d), lambda i: (i, 0)),
            pl.BlockSpec((d, d), lambda i: (0, 0)),
            pl.BlockSpec((1, d), lambda i: (0, 0)),
        ],
        out_specs=pl.BlockSpec((blk, d), lambda i: (i, 0)),
        out_shape=jax.ShapeDtypeStruct((n, d), jnp.float32),
    )(x, wcat, bcat)


def _scale_call(hcat, dparts, n, d):
    blk = 2000

    def body(h_ref, dp_ref, hs_ref):
        deg = jnp.sum(dp_ref[0], axis=1) + jnp.sum(dp_ref[1], axis=1)
        dinv = lax.rsqrt(jnp.maximum(deg, 1.0))
        hs_ref[...] = h_ref[...] * dinv[:, None]

    return pl.pallas_call(
        body,
        grid=(n // blk,),
        in_specs=[
            pl.BlockSpec((blk, d), lambda i: (i, 0)),
            pl.BlockSpec((2, blk, 16), lambda i: (0, i, 0)),
        ],
        out_specs=pl.BlockSpec((blk, d), lambda i: (i, 0)),
        out_shape=jax.ShapeDtypeStruct((n, d), jnp.float32),
    )(hcat, dparts)


def _post_call(parts, dparts, x, gamma, beta, w1, b1, w2, b2, n, d, mlp):
    blk = 2000

    def body(p_ref, dp_ref, x_ref, g_ref, be_ref,
             w1_ref, b1_ref, w2_ref, b2_ref, out_ref, xn_scr):
        deg = jnp.sum(dp_ref[0], axis=1) + jnp.sum(dp_ref[1], axis=1)
        dinv = lax.rsqrt(jnp.maximum(deg, 1.0))
        cat = (p_ref[0] + p_ref[1]) * dinv[:, None]
        m = jnp.mean(cat, axis=0)
        v = jnp.mean((cat - m[None, :]) ** 2, axis=0)
        scale = lax.rsqrt(v + 1e-5)
        xn_scr[...] = ((cat - m[None, :]) * scale[None, :]
                       * g_ref[...] + be_ref[...])
        w1b = w1_ref[...].astype(jnp.bfloat16)
        w2b = w2_ref[...].astype(jnp.bfloat16)

        def blk_body(i, carry):
            xb = xn_scr[pl.ds(i * blk, blk), :].astype(jnp.bfloat16)
            h1 = jax.nn.gelu(
                jnp.dot(xb, w1b, preferred_element_type=jnp.float32)
                + b1_ref[...])
            out_ref[pl.ds(i * blk, blk), :] = (
                jnp.dot(h1.astype(jnp.bfloat16), w2b,
                        preferred_element_type=jnp.float32)
                + b2_ref[...] + x_ref[pl.ds(i * blk, blk), :])
            return carry

        lax.fori_loop(0, n // blk, blk_body, 0)

    return pl.pallas_call(
        body,
        out_shape=jax.ShapeDtypeStruct((n, d), jnp.float32),
        scratch_shapes=[pltpu.VMEM((n, d), jnp.float32)],
    )(parts, dparts, x, gamma, beta, w1, b1, w2, b2)


def kernel(x, edge_index, W, b, bn_gamma, bn_beta, W1, b1, W2, b2):
    n, d = x.shape
    h = W.shape[0]
    mlp = W1.shape[1]
    e = edge_index.shape[1]
    assert e % (NW * CH) == 0 and e // (NW * CH) > 2 * IB and n % RCH == 0

    wcat = jnp.concatenate([W[i] for i in range(h)], axis=1)
    bcat = b.reshape(1, -1)
    ei_flat = edge_index.reshape(-1)
    zeros16 = jnp.zeros((n, 16), jnp.float32)
    zeros_nd = jnp.zeros((n, d), jnp.float32)

    dflat = _deg_call(ei_flat, zeros16, n, e)
    dparts = dflat.reshape(NC, n, 16)
    hcat = _hcat_call(x, wcat, bcat, n, d)
    hs = _scale_call(hcat, dparts, n, d)
    pflat = _edge_call(hs, ei_flat, zeros_nd, n, d, e)
    parts = pflat.reshape(NC, n, d)
    out = _post_call(parts, dparts, x, bn_gamma.reshape(1, -1),
                     bn_beta.reshape(1, -1), W1, b1.reshape(1, -1),
                     W2, b2.reshape(1, -1), n, d, mlp)
    return out


# speedup vs baseline: 1.0183x; 1.0183x over previous
import functools

import jax
import jax.numpy as jnp
from jax import lax
from jax.experimental import pallas as pl
from jax.experimental.pallas import tpu as pltpu
from jax.experimental.pallas import tpu_sc as plsc

NC = 2
NS = 16
NW = NC * NS
CH = 40
R = 6
IB = 2 * R
RCH = 400


def _sc_mesh():
    return plsc.VectorSubcoreMesh(core_axis_name="c", subcore_axis_name="s")


def _rowchunk_copy(s, n, copy_one):
    nchunk = n // RCH
    reps = (nchunk + NS - 1) // NS
    for rep in range(reps):
        cid = s + rep * NS
        if (rep + 1) * NS <= nchunk:
            copy_one(cid)
        else:
            @pl.when(cid < nchunk)
            def _():
                copy_one(cid)


def _deg_call(ei_flat, zeros16, n, e):
    CHD = 80
    P = 8
    epw = e // NW
    nch = epw // CHD
    ones_rows = jnp.concatenate(
        [jnp.ones((CHD, 1), jnp.float32), jnp.zeros((CHD, 15), jnp.float32)],
        axis=1)

    @functools.partial(
        pl.kernel,
        out_type=jax.ShapeDtypeStruct((NC * n, 16), jnp.float32),
        mesh=_sc_mesh(),
        scratch_types=(
            [pltpu.VMEM((2 * P, CHD), jnp.int32),
             pltpu.VMEM((CHD, 16), jnp.float32),
             pltpu.VMEM_SHARED((n, 16), jnp.float32)]
            + [pltpu.SemaphoreType.DMA] * (3 * P)
        ),
    )
    def deg_kernel(dst_hbm, ones_hbm, z16_hbm, out_hbm, di_v, ones_v, dacc,
                   *sems):
        sidx = sems[:2 * P]
        ssc = sems[2 * P:]
        c = lax.axis_index("c")
        s = lax.axis_index("s")
        wid = c * NS + s
        base = e + wid * epw
        _rowchunk_copy(s, n, lambda cid: pltpu.sync_copy(
            z16_hbm.at[pl.ds(cid * RCH, RCH)],
            dacc.at[pl.ds(cid * RCH, RCH)]))
        pltpu.sync_copy(ones_hbm, ones_v)
        plsc.subcore_barrier()

        def idx_start(j, slot):
            pltpu.async_copy(dst_hbm.at[pl.ds(base + j * CHD, CHD)],
                             di_v.at[slot], sidx[slot])

        def idx_wait(j, slot):
            pltpu.make_async_copy(dst_hbm.at[pl.ds(base + j * CHD, CHD)],
                                  di_v.at[slot], sidx[slot]).wait()

        def scat_start(slot, b):
            pltpu.async_copy(ones_v, dacc.at[di_v.at[slot]], ssc[b],
                             add=True)

        def scat_wait(slot, b):
            pltpu.make_async_copy(ones_v, dacc.at[di_v.at[slot]],
                                  ssc[b]).wait()

        def step(j, u, do_wait, do_issue):
            b = u % P
            if do_wait:
                scat_wait((u + P) % (2 * P), b)
            if do_issue:
                idx_start(j + P, (u + P) % (2 * P))
            idx_wait(j, u)
            scat_start(u, b)

        for i in range(P):
            idx_start(i, i)
        for j in range(P):
            step(j, j, False, True)

        def body(k, carry):
            for u2 in range(2 * P):
                j = P + k * 2 * P + u2
                step(j, (P + u2) % (2 * P), True, True)
            return carry

        n_main = (nch - 2 * P) // (2 * P)
        lax.fori_loop(0, n_main, body, 0)
        for j in range(P + n_main * 2 * P, nch):
            step(j, j % (2 * P), True, j + P < nch)
        for i in range(P):
            j = nch - P + i
            scat_wait(j % (2 * P), j % P)
        plsc.subcore_barrier()
        _rowchunk_copy(s, n, lambda cid: pltpu.sync_copy(
            dacc.at[pl.ds(cid * RCH, RCH)],
            out_hbm.at[pl.ds(c * n + cid * RCH, RCH)]))

    return deg_kernel(ei_flat, ones_rows, zeros16)


def _edge_call(hs, ei_flat, zeros, n, d, e):
    epw = e // NW
    nch = epw // CH

    @functools.partial(
        pl.kernel,
        out_type=jax.ShapeDtypeStruct((NC * n, d), jnp.float32),
        mesh=_sc_mesh(),
        scratch_types=(
            [pltpu.VMEM((IB, CH), jnp.int32),
             pltpu.VMEM((IB, CH), jnp.int32),
             pltpu.VMEM((R, CH, d), jnp.float32),
             pltpu.VMEM_SHARED((n, d), jnp.float32)]
            + [pltpu.SemaphoreType.DMA] * (IB + R)
        ),
    )
    def edge_kernel(hs_hbm, ei_hbm, z_hbm, out_hbm,
                    si_v, di_v, rows_v, acc, *sems):
        sidx = sems[:IB]
        srow = sems[IB:]
        c = lax.axis_index("c")
        s = lax.axis_index("s")
        wid = c * NS + s
        sbase = wid * epw
        dbase = e + wid * epw
        _rowchunk_copy(s, n, lambda cid: pltpu.sync_copy(
            z_hbm.at[pl.ds(cid * RCH, RCH)],
            acc.at[pl.ds(cid * RCH, RCH)]))
        plsc.subcore_barrier()

        def idx_start(j, u):
            pltpu.async_copy(ei_hbm.at[pl.ds(sbase + j * CH, CH)],
                             si_v.at[u], sidx[u])
            pltpu.async_copy(ei_hbm.at[pl.ds(dbase + j * CH, CH)],
                             di_v.at[u], sidx[u])

        def idx_wait(j, u):
            pltpu.make_async_copy(ei_hbm.at[pl.ds(sbase + j * CH, CH)],
                                  si_v.at[u], sidx[u]).wait()
            pltpu.make_async_copy(ei_hbm.at[pl.ds(dbase + j * CH, CH)],
                                  di_v.at[u], sidx[u]).wait()

        def gather_start(islot, b):
            pltpu.async_copy(hs_hbm.at[si_v.at[islot]], rows_v.at[b], srow[b])

        def gather_wait(islot, b):
            pltpu.make_async_copy(hs_hbm.at[si_v.at[islot]], rows_v.at[b],
                                  srow[b]).wait()

        def scat(islot, b):
            pltpu.sync_copy(rows_v.at[b], acc.at[di_v.at[islot]], add=True)

        for i in range(IB):
            idx_start(i, i)
        for b in range(R):
            idx_wait(b, b)
            gather_start(b, b)

        def step(j, u, do_idx, do_gather):
            b = u % R
            islot = u % IB
            islot2 = (u + R) % IB
            gather_wait(islot, b)
            scat(islot, b)
            if do_idx:
                idx_start(j + IB, islot)
            if do_gather:
                idx_wait(j + R, islot2)
                gather_start(islot2, b)

        def body(k, carry):
            for u in range(IB):
                step(k * IB + u, u, True, True)
            return carry

        n_main = (nch - IB) // IB
        lax.fori_loop(0, n_main, body, 0)
        for j in range(n_main * IB, nch):
            step(j, j % IB, j + IB < nch, j + R < nch)

        plsc.subcore_barrier()
        _rowchunk_copy(s, n, lambda cid: pltpu.sync_copy(
            acc.at[pl.ds(cid * RCH, RCH)],
            out_hbm.at[pl.ds(c * n + cid * RCH, RCH)]))

    return edge_kernel(hs, ei_flat, zeros)


def _mm_call(x, wcat, bcat, dparts, n, d):
    blk = 2000

    def body(x_ref, w_ref, b_ref, dp_ref, hs_ref):
        deg = jnp.sum(dp_ref[0], axis=1) + jnp.sum(dp_ref[1], axis=1)
        dinv = lax.rsqrt(jnp.maximum(deg, 1.0))
        h = jnp.dot(x_ref[...], w_ref[...],
                    preferred_element_type=jnp.float32) + b_ref[...]
        hs_ref[...] = h * dinv[:, None]

    return pl.pallas_call(
        body,
        grid=(n // blk,),
        in_specs=[
            pl.BlockSpec((blk, d), lambda i: (i, 0)),
            pl.BlockSpec((d, d), lambda i: (0, 0)),
            pl.BlockSpec((1, d), lambda i: (0, 0)),
            pl.BlockSpec((2, blk, 16), lambda i: (0, i, 0)),
        ],
        out_specs=pl.BlockSpec((blk, d), lambda i: (i, 0)),
        out_shape=jax.ShapeDtypeStruct((n, d), jnp.float32),
    )(x, wcat, bcat, dparts)


def _post_call(parts, dparts, x, gamma, beta, w1, b1, w2, b2, n, d, mlp):
    blk = 2000

    def body(p_ref, dp_ref, x_ref, g_ref, be_ref,
             w1_ref, b1_ref, w2_ref, b2_ref, out_ref, xn_scr):
        deg = jnp.sum(dp_ref[0], axis=1) + jnp.sum(dp_ref[1], axis=1)
        dinv = lax.rsqrt(jnp.maximum(deg, 1.0))
        cat = (p_ref[0] + p_ref[1]) * dinv[:, None]
        m = jnp.mean(cat, axis=0)
        v = jnp.mean((cat - m[None, :]) ** 2, axis=0)
        scale = lax.rsqrt(v + 1e-5)
        xn_scr[...] = ((cat - m[None, :]) * scale[None, :]
                       * g_ref[...] + be_ref[...])
        w1b = w1_ref[...].astype(jnp.bfloat16)
        w2b = w2_ref[...].astype(jnp.bfloat16)

        def blk_body(i, carry):
            xb = xn_scr[pl.ds(i * blk, blk), :].astype(jnp.bfloat16)
            h1 = jax.nn.gelu(
                jnp.dot(xb, w1b, preferred_element_type=jnp.float32)
                + b1_ref[...])
            out_ref[pl.ds(i * blk, blk), :] = (
                jnp.dot(h1.astype(jnp.bfloat16), w2b,
                        preferred_element_type=jnp.float32)
                + b2_ref[...] + x_ref[pl.ds(i * blk, blk), :])
            return carry

        lax.fori_loop(0, n // blk, blk_body, 0)

    return pl.pallas_call(
        body,
        out_shape=jax.ShapeDtypeStruct((n, d), jnp.float32),
        scratch_shapes=[pltpu.VMEM((n, d), jnp.float32)],
    )(parts, dparts, x, gamma, beta, w1, b1, w2, b2)


def kernel(x, edge_index, W, b, bn_gamma, bn_beta, W1, b1, W2, b2):
    n, d = x.shape
    h = W.shape[0]
    mlp = W1.shape[1]
    e = edge_index.shape[1]
    assert e % (NW * CH) == 0 and e // (NW * CH) > 2 * IB and n % RCH == 0

    wcat = jnp.concatenate([W[i] for i in range(h)], axis=1)
    bcat = b.reshape(1, -1)
    ei_flat = edge_index.reshape(-1)
    zeros16 = jnp.zeros((n, 16), jnp.float32)
    zeros_nd = jnp.zeros((n, d), jnp.float32)

    dflat = _deg_call(ei_flat, zeros16, n, e)
    dparts = dflat.reshape(NC, n, 16)
    hs = _mm_call(x, wcat, bcat, dparts, n, d)
    pflat = _edge_call(hs, ei_flat, zeros_nd, n, d, e)
    parts = pflat.reshape(NC, n, d)
    out = _post_call(parts, dparts, x, bn_gamma.reshape(1, -1),
                     bn_beta.reshape(1, -1), W1, b1.reshape(1, -1),
                     W2, b2.reshape(1, -1), n, d, mlp)
    return out


# speedup vs baseline: 1.0193x; 1.0011x over previous
import functools

import jax
import jax.numpy as jnp
from jax import lax
from jax.experimental import pallas as pl
from jax.experimental.pallas import tpu as pltpu
from jax.experimental.pallas import tpu_sc as plsc

NC = 2
NS = 16
NW = NC * NS
CH = 40
R = 6
IB = 2 * R
RCH = 400


def _sc_mesh():
    return plsc.VectorSubcoreMesh(core_axis_name="c", subcore_axis_name="s")


def _rowchunk_copy(s, n, copy_one):
    nchunk = n // RCH
    reps = (nchunk + NS - 1) // NS
    for rep in range(reps):
        cid = s + rep * NS
        if (rep + 1) * NS <= nchunk:
            copy_one(cid)
        else:
            @pl.when(cid < nchunk)
            def _():
                copy_one(cid)


def _deg_call(ei_flat, zeros16, n, e):
    CHD = 80
    P = 8
    epw = e // NW
    nch = epw // CHD
    ones_rows = jnp.concatenate(
        [jnp.ones((CHD, 1), jnp.float32), jnp.zeros((CHD, 15), jnp.float32)],
        axis=1)

    @functools.partial(
        pl.kernel,
        out_type=jax.ShapeDtypeStruct((NC * n, 16), jnp.float32),
        mesh=_sc_mesh(),
        scratch_types=(
            [pltpu.VMEM((2 * P, CHD), jnp.int32),
             pltpu.VMEM((CHD, 16), jnp.float32),
             pltpu.VMEM_SHARED((n, 16), jnp.float32)]
            + [pltpu.SemaphoreType.DMA] * (3 * P)
        ),
    )
    def deg_kernel(dst_hbm, ones_hbm, z16_hbm, out_hbm, di_v, ones_v, dacc,
                   *sems):
        sidx = sems[:2 * P]
        ssc = sems[2 * P:]
        c = lax.axis_index("c")
        s = lax.axis_index("s")
        wid = c * NS + s
        base = e + wid * epw
        _rowchunk_copy(s, n, lambda cid: pltpu.sync_copy(
            z16_hbm.at[pl.ds(cid * RCH, RCH)],
            dacc.at[pl.ds(cid * RCH, RCH)]))
        pltpu.sync_copy(ones_hbm, ones_v)
        plsc.subcore_barrier()

        def idx_start(j, slot):
            pltpu.async_copy(dst_hbm.at[pl.ds(base + j * CHD, CHD)],
                             di_v.at[slot], sidx[slot])

        def idx_wait(j, slot):
            pltpu.make_async_copy(dst_hbm.at[pl.ds(base + j * CHD, CHD)],
                                  di_v.at[slot], sidx[slot]).wait()

        def scat_start(slot, b):
            pltpu.async_copy(ones_v, dacc.at[di_v.at[slot]], ssc[b],
                             add=True)

        def scat_wait(slot, b):
            pltpu.make_async_copy(ones_v, dacc.at[di_v.at[slot]],
                                  ssc[b]).wait()

        def step(j, u, do_wait, do_issue):
            b = u % P
            if do_wait:
                scat_wait((u + P) % (2 * P), b)
            if do_issue:
                idx_start(j + P, (u + P) % (2 * P))
            idx_wait(j, u)
            scat_start(u, b)

        for i in range(P):
            idx_start(i, i)
        for j in range(P):
            step(j, j, False, True)

        def body(k, carry):
            for u2 in range(2 * P):
                j = P + k * 2 * P + u2
                step(j, (P + u2) % (2 * P), True, True)
            return carry

        n_main = (nch - 2 * P) // (2 * P)
        lax.fori_loop(0, n_main, body, 0)
        for j in range(P + n_main * 2 * P, nch):
            step(j, j % (2 * P), True, j + P < nch)
        for i in range(P):
            j = nch - P + i
            scat_wait(j % (2 * P), j % P)
        plsc.subcore_barrier()
        _rowchunk_copy(s, n, lambda cid: pltpu.sync_copy(
            dacc.at[pl.ds(cid * RCH, RCH)],
            out_hbm.at[pl.ds(c * n + cid * RCH, RCH)]))

    return deg_kernel(ei_flat, ones_rows, zeros16)


def _edge_call(hs, ei_flat, zeros, n, d, e):
    epw = e // NW
    nch = epw // CH

    @functools.partial(
        pl.kernel,
        out_type=jax.ShapeDtypeStruct((NC * n, d), jnp.float32),
        mesh=_sc_mesh(),
        scratch_types=(
            [pltpu.VMEM((IB, CH), jnp.int32),
             pltpu.VMEM((IB, CH), jnp.int32),
             pltpu.VMEM((R, CH, d), jnp.float32),
             pltpu.VMEM_SHARED((n, d), jnp.float32)]
            + [pltpu.SemaphoreType.DMA] * (IB + R)
        ),
    )
    def edge_kernel(hs_hbm, ei_hbm, z_hbm, out_hbm,
                    si_v, di_v, rows_v, acc, *sems):
        sidx = sems[:IB]
        srow = sems[IB:]
        c = lax.axis_index("c")
        s = lax.axis_index("s")
        wid = c * NS + s
        sbase = wid * epw
        dbase = e + wid * epw
        _rowchunk_copy(s, n, lambda cid: pltpu.sync_copy(
            z_hbm.at[pl.ds(cid * RCH, RCH)],
            acc.at[pl.ds(cid * RCH, RCH)]))
        plsc.subcore_barrier()

        def idx_start(j, u):
            pltpu.async_copy(ei_hbm.at[pl.ds(sbase + j * CH, CH)],
                             si_v.at[u], sidx[u])
            pltpu.async_copy(ei_hbm.at[pl.ds(dbase + j * CH, CH)],
                             di_v.at[u], sidx[u])

        def idx_wait(j, u):
            pltpu.make_async_copy(ei_hbm.at[pl.ds(sbase + j * CH, CH)],
                                  si_v.at[u], sidx[u]).wait()
            pltpu.make_async_copy(ei_hbm.at[pl.ds(dbase + j * CH, CH)],
                                  di_v.at[u], sidx[u]).wait()

        def gather_start(islot, b):
            pltpu.async_copy(hs_hbm.at[si_v.at[islot]], rows_v.at[b], srow[b])

        def gather_wait(islot, b):
            pltpu.make_async_copy(hs_hbm.at[si_v.at[islot]], rows_v.at[b],
                                  srow[b]).wait()

        def scat(islot, b):
            pltpu.sync_copy(rows_v.at[b], acc.at[di_v.at[islot]], add=True)

        for i in range(IB):
            idx_start(i, i)
        for b in range(R):
            idx_wait(b, b)
            gather_start(b, b)

        def step(j, u, do_idx, do_gather):
            b = u % R
            islot = u % IB
            islot2 = (u + R) % IB
            gather_wait(islot, b)
            scat(islot, b)
            if do_idx:
                idx_start(j + IB, islot)
            if do_gather:
                idx_wait(j + R, islot2)
                gather_start(islot2, b)

        def body(k, carry):
            for u in range(IB):
                step(k * IB + u, u, True, True)
            return carry

        n_main = (nch - IB) // IB
        lax.fori_loop(0, n_main, body, 0)
        for j in range(n_main * IB, nch):
            step(j, j % IB, j + IB < nch, j + R < nch)

        plsc.subcore_barrier()
        _rowchunk_copy(s, n, lambda cid: pltpu.sync_copy(
            acc.at[pl.ds(cid * RCH, RCH)],
            out_hbm.at[pl.ds(c * n + cid * RCH, RCH)]))

    return edge_kernel(hs, ei_flat, zeros)


def _mm_call(x, wcat, bcat, dparts, n, d):
    blk = 2000

    def body(x_ref, w_ref, b_ref, dp_ref, hs_ref):
        deg = jnp.sum(dp_ref[0], axis=1) + jnp.sum(dp_ref[1], axis=1)
        dinv = lax.rsqrt(jnp.maximum(deg, 1.0))
        h = jnp.dot(x_ref[...], w_ref[...],
                    preferred_element_type=jnp.float32) + b_ref[...]
        hs_ref[...] = h * dinv[:, None]

    return pl.pallas_call(
        body,
        grid=(n // blk,),
        in_specs=[
            pl.BlockSpec((blk, d), lambda i: (i, 0)),
            pl.BlockSpec((d, d), lambda i: (0, 0)),
            pl.BlockSpec((1, d), lambda i: (0, 0)),
            pl.BlockSpec((2, blk, 16), lambda i: (0, i, 0)),
        ],
        out_specs=pl.BlockSpec((blk, d), lambda i: (i, 0)),
        out_shape=jax.ShapeDtypeStruct((n, d), jnp.float32),
    )(x, wcat, bcat, dparts)


def _post_call(parts, dparts, x, gamma, beta, w1, b1, w2, b2, n, d, mlp):
    blk = 1000
    nb = n // blk

    def body(p_ref, dp_ref, x_ref, g_ref, be_ref,
             w1_ref, b1_ref, w2_ref, b2_ref, out_ref, cat_scr, st_scr):
        ph = pl.program_id(0)
        i = pl.program_id(1)

        @pl.when(ph == 0)
        def _():
            deg = jnp.sum(dp_ref[0], axis=1) + jnp.sum(dp_ref[1], axis=1)
            dinv = lax.rsqrt(jnp.maximum(deg, 1.0))
            cat = (p_ref[0] + p_ref[1]) * dinv[:, None]
            cat_scr[pl.ds(i * blk, blk), :] = cat
            s1 = jnp.sum(cat, axis=0)[None, :]
            s2 = jnp.sum(cat * cat, axis=0)[None, :]
            acc = jnp.concatenate([s1, s2], axis=0)

            @pl.when(i == 0)
            def _():
                st_scr[...] = acc

            @pl.when(i != 0)
            def _():
                st_scr[...] += acc

        @pl.when(ph == 1)
        def _():
            m = st_scr[0:1, :] * (1.0 / n)
            var = st_scr[1:2, :] * (1.0 / n) - m * m
            cat = cat_scr[pl.ds(i * blk, blk), :]
            xn = (cat - m) * lax.rsqrt(var + 1e-5) * g_ref[...] + be_ref[...]
            h1 = jax.nn.gelu(
                jnp.dot(xn.astype(jnp.bfloat16),
                        w1_ref[...].astype(jnp.bfloat16),
                        preferred_element_type=jnp.float32) + b1_ref[...])
            out_ref[...] = (jnp.dot(h1.astype(jnp.bfloat16),
                                    w2_ref[...].astype(jnp.bfloat16),
                                    preferred_element_type=jnp.float32)
                            + b2_ref[...] + x_ref[...])

    return pl.pallas_call(
        body,
        grid=(2, nb),
        in_specs=[
            pl.BlockSpec((2, blk, d),
                         lambda p, i: (0, jnp.where(p == 0, i, 0), 0)),
            pl.BlockSpec((2, blk, 16),
                         lambda p, i: (0, jnp.where(p == 0, i, 0), 0)),
            pl.BlockSpec((blk, d), lambda p, i: (jnp.where(p == 1, i, 0), 0)),
            pl.BlockSpec((1, d), lambda p, i: (0, 0)),
            pl.BlockSpec((1, d), lambda p, i: (0, 0)),
            pl.BlockSpec((d, mlp), lambda p, i: (0, 0)),
            pl.BlockSpec((1, mlp), lambda p, i: (0, 0)),
            pl.BlockSpec((mlp, d), lambda p, i: (0, 0)),
            pl.BlockSpec((1, d), lambda p, i: (0, 0)),
        ],
        out_specs=pl.BlockSpec((blk, d),
                               lambda p, i: (jnp.where(p == 1, i, 0), 0)),
        out_shape=jax.ShapeDtypeStruct((n, d), jnp.float32),
        scratch_shapes=[pltpu.VMEM((n, d), jnp.float32),
                        pltpu.VMEM((2, d), jnp.float32)],
    )(parts, dparts, x, gamma, beta, w1, b1, w2, b2)


def kernel(x, edge_index, W, b, bn_gamma, bn_beta, W1, b1, W2, b2):
    n, d = x.shape
    h = W.shape[0]
    mlp = W1.shape[1]
    e = edge_index.shape[1]
    assert e % (NW * CH) == 0 and e // (NW * CH) > 2 * IB and n % RCH == 0

    wcat = jnp.concatenate([W[i] for i in range(h)], axis=1)
    bcat = b.reshape(1, -1)
    ei_flat = edge_index.reshape(-1)
    zeros16 = jnp.zeros((n, 16), jnp.float32)
    zeros_nd = jnp.zeros((n, d), jnp.float32)

    dflat = _deg_call(ei_flat, zeros16, n, e)
    dparts = dflat.reshape(NC, n, 16)
    hs = _mm_call(x, wcat, bcat, dparts, n, d)
    pflat = _edge_call(hs, ei_flat, zeros_nd, n, d, e)
    parts = pflat.reshape(NC, n, d)
    out = _post_call(parts, dparts, x, bn_gamma.reshape(1, -1),
                     bn_beta.reshape(1, -1), W1, b1.reshape(1, -1),
                     W2, b2.reshape(1, -1), n, d, mlp)
    return out
